# baseline (speedup 1.0000x reference)
import functools

import jax
import jax.numpy as jnp
from jax import lax
from jax.experimental import pallas as pl
from jax.experimental.pallas import tpu as pltpu
import jax.experimental.pallas.tpu_sc as plsc

ROWS = 8192
DIM = 1024
NUM_CORES = 2
NUM_SUBCORES = 16
NUM_WORKERS = NUM_CORES * NUM_SUBCORES
ROWS_PER_WORKER = ROWS // NUM_WORKERS
CHUNK = 56
CHUNKS = [CHUNK] * (ROWS_PER_WORKER // CHUNK)
if ROWS_PER_WORKER % CHUNK:
    CHUNKS.append(ROWS_PER_WORKER % CHUNK)
OFFS = [sum(CHUNKS[:i]) for i in range(len(CHUNKS))]
NCHUNKS = len(CHUNKS)
NBUF = 2


@functools.partial(
    pl.kernel,
    out_type=jax.ShapeDtypeStruct((ROWS, DIM), jnp.float32),
    mesh=plsc.VectorSubcoreMesh(core_axis_name="c", subcore_axis_name="s"),
    scratch_types=(
        [pltpu.VMEM_SHARED((NUM_SUBCORES, NBUF, CHUNK, DIM), jnp.float32)]
        + [pltpu.SemaphoreType.DMA] * (2 * NBUF)
    ),
)
def _pe_lookup(pe_hbm, out_hbm, sbuf, *sems):
    sid = lax.axis_index("s")
    wid = sid * NUM_CORES + lax.axis_index("c")
    base = wid * ROWS_PER_WORKER
    buf = sbuf.at[sid]
    gsems = sems[:NBUF]
    ssems = sems[NBUF:]

    def issue_gather(i):
        return pltpu.async_copy(
            pe_hbm.at[pl.ds(base + OFFS[i], CHUNKS[i])],
            buf.at[i % NBUF, pl.ds(0, CHUNKS[i])],
            gsems[i % NBUF])

    def issue_scatter(i):
        return pltpu.async_copy(
            buf.at[i % NBUF, pl.ds(0, CHUNKS[i])],
            out_hbm.at[pl.ds(base + OFFS[i], CHUNKS[i])],
            ssems[i % NBUF])

    gath = [None] * NCHUNKS
    scat = [None] * NCHUNKS
    gath[0] = issue_gather(0)
    for i in range(NCHUNKS):
        if i + 1 < NCHUNKS:
            if i + 1 >= NBUF:
                scat[i + 1 - NBUF].wait()
            gath[i + 1] = issue_gather(i + 1)
        gath[i].wait()
        scat[i] = issue_scatter(i)
    for i in range(max(0, NCHUNKS - NBUF), NCHUNKS):
        scat[i].wait()


def kernel(x, pe):
    del x
    return _pe_lookup(pe)


# speedup vs baseline: 1.7546x; 1.7546x over previous
import jax
import jax.numpy as jnp
from jax.experimental import pallas as pl
from jax.experimental.pallas import tpu as pltpu

ROWS = 8192
DIM = 1024
BLK = 512


def _copy_body(pe_ref, o_ref):
    o_ref[...] = pe_ref[...]


def kernel(x, pe):
    del x
    return pl.pallas_call(
        _copy_body,
        grid=(ROWS // BLK,),
        in_specs=[pl.BlockSpec((BLK, DIM), lambda i: (i, 0))],
        out_specs=pl.BlockSpec((BLK, DIM), lambda i: (i, 0)),
        out_shape=jax.ShapeDtypeStruct((ROWS, DIM), jnp.float32),
    )(pe)
